# baseline (speedup 1.0000x reference)
import dataclasses

import jax
import jax.numpy as jnp
from jax import lax
from jax.experimental import pallas as pl
from jax.experimental.pallas import tpu as pltpu
from jax.experimental.pallas import tpu_sc as plsc

B = 16384
D = 64
L = 16
NC, NS = 2, 16
NW = NC * NS
BPW = B // NW
K = 128
NCHUNK = BPW // K


def _body(item_vec_hbm, user_vec_hbm, item_table_hbm, user_table_hbm,
          out_hbm, iidx_v, uidx_v, irows_v, urows_v, out_v):
    wid = lax.axis_index("s") * NC + lax.axis_index("c")
    base = wid * BPW
    pltpu.sync_copy(item_vec_hbm.at[pl.ds(base, BPW)], iidx_v)
    pltpu.sync_copy(user_vec_hbm.at[pl.ds(base, BPW)], uidx_v)

    lane = lax.iota(jnp.int32, L)

    for c in range(NCHUNK):
        pltpu.sync_copy(item_table_hbm.at[iidx_v.at[pl.ds(c * K, K)]], irows_v)
        pltpu.sync_copy(user_table_hbm.at[uidx_v.at[pl.ds(c * K, K)]], urows_v)

        @pl.loop(0, K // L)
        def _(g):
            o = jnp.zeros((L,), jnp.float32)
            for r in range(L):
                row = g * L + r
                s16 = irows_v[row, pl.ds(0, L)] * urows_v[row, pl.ds(0, L)]
                for j in range(1, D // L):
                    s16 = s16 + (irows_v[row, pl.ds(j * L, L)]
                                 * urows_v[row, pl.ds(j * L, L)])
                o = jnp.where(lane == r, jnp.sum(s16), o)
            out_v[pl.ds(c * K + g * L, L)] = o

    pltpu.sync_copy(out_v, out_hbm.at[pl.ds(base, BPW)])


def kernel(item_vec, user_vec, item_table, user_table):
    mesh = plsc.VectorSubcoreMesh(core_axis_name="c", subcore_axis_name="s")
    cp = pltpu.CompilerParams(
        needs_layout_passes=False, use_tc_tiling_on_sc=False)
    f = pl.kernel(
        _body,
        out_type=jax.ShapeDtypeStruct((B,), jnp.float32),
        mesh=mesh,
        compiler_params=cp,
        scratch_types=[
            pltpu.VMEM((BPW,), jnp.int32),
            pltpu.VMEM((BPW,), jnp.int32),
            pltpu.VMEM((K, D), jnp.float32),
            pltpu.VMEM((K, D), jnp.float32),
            pltpu.VMEM((BPW,), jnp.float32),
        ],
    )
    return f(item_vec, user_vec, item_table, user_table)
